# baseline (speedup 1.0000x reference)
import functools

import jax
import jax.numpy as jnp
from jax import lax
from jax.experimental import pallas as pl
from jax.experimental.pallas import tpu as pltpu
from jax.experimental.pallas import tpu_sc as plsc

_B = 4096
_SD = 200 * 64
_LANES = 16
_NW = 32
_RPW = _SD // _NW
_CH = 8
_NB = 2
_NCHUNK = _RPW // _CH
_UNROLL = 8


def _make_sc_kernel():
    mesh = plsc.VectorSubcoreMesh(core_axis_name="c", subcore_axis_name="s")

    @functools.partial(
        pl.kernel,
        mesh=mesh,
        out_type=jax.ShapeDtypeStruct((_SD, _B), jnp.float32),
        scratch_types=[
            pltpu.VMEM((_RPW * _LANES,), jnp.float32),
            pltpu.VMEM((_NB, _CH, _B), jnp.float32),
            pltpu.SemaphoreType.DMA((_NB,)),
            pltpu.SemaphoreType.DMA((_NB,)),
        ],
    )
    def sc_add(x_hbm, posb_hbm, out_hbm, posv, buf, in_sems, out_sems):
        wid = lax.axis_index("s") * 2 + lax.axis_index("c")
        base = wid * _RPW

        pltpu.sync_copy(
            posb_hbm.at[pl.ds(base * _LANES, _RPW * _LANES)], posv
        )

        for b in range(_NB):
            pltpu.async_copy(
                x_hbm.at[pl.ds(base + b * _CH, _CH)], buf.at[b], in_sems.at[b]
            )

        def outer(i, carry):
            for b in range(_NB):
                k = i * _NB + b
                row0 = base + k * _CH
                pltpu.make_async_copy(
                    x_hbm.at[pl.ds(row0, _CH)], buf.at[b], in_sems.at[b]
                ).wait()

                for r in range(_CH):
                    rowref = buf.at[b, r]
                    psplat = posv[pl.ds((k * _CH + r) * _LANES, _LANES)]

                    @plsc.parallel_loop(0, _B, step=_LANES, unroll=_UNROLL)
                    def _add(c):
                        sl = pl.ds(c, _LANES)
                        rowref[sl] = rowref[sl] + psplat

                pltpu.async_copy(
                    buf.at[b], out_hbm.at[pl.ds(row0, _CH)], out_sems.at[b]
                )

                @pl.when(i < _NCHUNK // _NB - 1)
                def _prefetch():
                    pltpu.make_async_copy(
                        buf.at[b], out_hbm.at[pl.ds(row0, _CH)], out_sems.at[b]
                    ).wait()
                    pltpu.async_copy(
                        x_hbm.at[pl.ds(row0 + _NB * _CH, _CH)],
                        buf.at[b],
                        in_sems.at[b],
                    )

            return carry

        lax.fori_loop(0, _NCHUNK // _NB, outer, 0)

        for b in range(_NB):
            row0 = base + ((_NCHUNK // _NB - 1) * _NB + b) * _CH
            pltpu.make_async_copy(
                buf.at[b], out_hbm.at[pl.ds(row0, _CH)], out_sems.at[b]
            ).wait()

    return sc_add


def kernel(x, pos_table):
    B, S, D = x.shape
    xt = x.transpose(1, 2, 0).reshape(S * D, B)
    posb = jnp.repeat(pos_table[:S].reshape(S * D), _LANES)
    out_t = _make_sc_kernel()(xt, posb)
    return out_t.reshape(S, D, B).transpose(2, 0, 1)


# speedup vs baseline: 1.2771x; 1.2771x over previous
import jax
import jax.numpy as jnp
from jax.experimental import pallas as pl
from jax.experimental.pallas import tpu as pltpu

_B = 4096
_SD = 200 * 64
_CH = 512
_Q = 4
_NCH = _SD // _CH
_CHQ = _CH // _Q


def _body(pos_ref, x_hbm, out_hbm, buf, obuf, in_sems, out_sems):
    i = pl.program_id(0)
    slot = jax.lax.rem(i, 2)
    nslot = jax.lax.rem(i + 1, 2)

    def in_copy(chunk, s, q):
        return pltpu.make_async_copy(
            x_hbm.at[pl.ds(chunk * _CH + q * _CHQ, _CHQ)],
            buf.at[s, pl.ds(q * _CHQ, _CHQ)],
            in_sems.at[s, q],
        )

    def out_copy(chunk, s, q):
        return pltpu.make_async_copy(
            obuf.at[s, pl.ds(q * _CHQ, _CHQ)],
            out_hbm.at[pl.ds(chunk * _CH + q * _CHQ, _CHQ)],
            out_sems.at[s, q],
        )

    @pl.when(i == 0)
    def _prologue():
        for q in range(_Q):
            in_copy(0, 0, q).start()

    @pl.when(i + 1 < _NCH)
    def _prefetch():
        for q in range(_Q):
            in_copy(i + 1, nslot, q).start()

    for q in range(_Q):
        in_copy(i, slot, q).wait()

    @pl.when(i >= 2)
    def _free_out():
        for q in range(_Q):
            out_copy(i - 2, slot, q).wait()

    obuf[slot] = buf[slot] + pos_ref[...]

    for q in range(_Q):
        out_copy(i, slot, q).start()

    @pl.when(i == _NCH - 1)
    def _drain():
        for q in range(_Q):
            out_copy(i - 1, nslot, q).wait()
        for q in range(_Q):
            out_copy(i, slot, q).wait()


def kernel(x, pos_table):
    B, S, D = x.shape
    xt = x.transpose(1, 2, 0).reshape(S * D, B)
    post = pos_table[:S].reshape(S * D, 1)
    out_t = pl.pallas_call(
        _body,
        grid=(_NCH,),
        in_specs=[
            pl.BlockSpec((_CH, 1), lambda i: (i, 0)),
            pl.BlockSpec(memory_space=pl.ANY),
        ],
        out_specs=pl.BlockSpec(memory_space=pl.ANY),
        out_shape=jax.ShapeDtypeStruct((S * D, B), jnp.float32),
        scratch_shapes=[
            pltpu.VMEM((2, _CH, _B), jnp.float32),
            pltpu.VMEM((2, _CH, _B), jnp.float32),
            pltpu.SemaphoreType.DMA((2, _Q)),
            pltpu.SemaphoreType.DMA((2, _Q)),
        ],
        compiler_params=pltpu.CompilerParams(
            dimension_semantics=("arbitrary",),
        ),
    )(post, xt)
    return out_t.reshape(S, D, B).transpose(2, 0, 1)


# speedup vs baseline: 1.2845x; 1.0059x over previous
import jax
import jax.numpy as jnp
from jax.experimental import pallas as pl
from jax.experimental.pallas import tpu as pltpu

_B = 4096
_SD = 200 * 64
_BLK = 800


def _add_body(x_ref, pos_ref, out_ref):
    out_ref[...] = x_ref[...] + pos_ref[...]


def kernel(x, pos_table):
    B, S, D = x.shape
    xt = x.transpose(1, 2, 0).reshape(S * D, B)
    post = pos_table[:S].reshape(S * D, 1)
    out_t = pl.pallas_call(
        _add_body,
        grid=(S * D // _BLK,),
        in_specs=[
            pl.BlockSpec((_BLK, B), lambda i: (i, 0)),
            pl.BlockSpec((_BLK, 1), lambda i: (i, 0)),
        ],
        out_specs=pl.BlockSpec((_BLK, B), lambda i: (i, 0)),
        out_shape=jax.ShapeDtypeStruct((S * D, B), x.dtype),
        compiler_params=pltpu.CompilerParams(skip_device_barrier=True),
    )(xt, post)
    return out_t.reshape(S, D, B).transpose(2, 0, 1)
